# baseline (speedup 1.0000x reference)
import functools

import jax
import jax.numpy as jnp
from jax import lax
from jax.experimental import pallas as pl
from jax.experimental.pallas import tpu as pltpu
from jax.experimental.pallas import tpu_sc as plsc

_CHUNK = 128
_K = 4
_PAD_COLS = 16
_FRAC0 = 0.5


def _node_rows(n):
    return ((n + 1 + 31) // 32) * 32


def _prep_edges(src, dst, n, R, frac0):
    E = src.shape[0]
    mesh = plsc.VectorSubcoreMesh(core_axis_name="c", subcore_axis_name="s")
    NC, NS = mesh.num_cores, mesh.num_subcores
    C = -(-E // _CHUNK)
    cpt0 = -(-int(C * frac0) // (NS * _K)) * _K
    cpt1 = max(_K, -(-(C - NS * cpt0) // (NS * _K)) * _K)
    skip1 = NS * cpt0 >= C
    halves = 2 if cpt0 > 144 else 1
    cpt0 = -(-cpt0 // (halves * _K)) * (halves * _K)
    E_pad = NS * (cpt0 + cpt1) * _CHUNK
    n_dummy = E_pad - E
    src = jnp.concatenate([src, jnp.zeros((n_dummy,), jnp.int32)])
    dst = jnp.concatenate(
        [dst, n + (jnp.arange(n_dummy, dtype=jnp.int32) % (R - n))])
    e0 = NS * cpt0 * _CHUNK
    c0_real = min(C, NS * cpt0)
    return dict(
        src0=src[:e0].reshape(NS, cpt0, _CHUNK),
        dst0=dst[:e0].reshape(NS, cpt0, _CHUNK),
        src1=src[e0:].reshape(NS, cpt1, _CHUNK),
        dst1=dst[e0:].reshape(NS, cpt1, _CHUNK),
        cpt0=cpt0, cpt1=cpt1, c0_real=c0_real, c1_real=C - c0_real,
        skip1=skip1, halves=halves, mesh=mesh, NC=NC, NS=NS,
    )


def _sc_segsum(y, ep, n):
    R, D = y.shape
    mesh, NC, NS = ep["mesh"], ep["NC"], ep["NS"]
    K = _K
    cpt0, cpt1 = ep["cpt0"], ep["cpt1"]
    c0_real, c1_real = ep["c0_real"], ep["c1_real"]
    skip1, halves = ep["skip1"], ep["halves"]
    ch = cpt0 // halves
    zeros = jnp.zeros((R // NS, D), jnp.float32)
    rpt = R // NS

    @functools.partial(
        pl.kernel,
        out_type=jax.ShapeDtypeStruct((NC, R, D), jnp.float32),
        mesh=mesh,
        scratch_types=[
            pltpu.VMEM((ch, _CHUNK), jnp.int32),
            pltpu.VMEM((ch, _CHUNK), jnp.int32),
            pltpu.VMEM((K, _CHUNK, D), jnp.float32),
            pltpu.VMEM_SHARED((R, D), jnp.float32),
            pltpu.SemaphoreType.DMA,
            pltpu.SemaphoreType.DMA,
        ],
        compiler_params=pltpu.CompilerParams(use_tc_tiling_on_sc=False),
    )
    def seg_kernel(y_hbm, src0_hbm, dst0_hbm, src1_hbm, dst1_hbm, z_hbm,
                   out_hbm, sidx, didx, rows, agg, sem, ssem):
        cid = lax.axis_index("c")
        sid = lax.axis_index("s")
        base = sid * rpt
        on0 = cid == 0

        nblk = jnp.where(on0, 1, 0 if skip1 else 1)

        def zbody(i, carry):
            pltpu.sync_copy(z_hbm, agg.at[pl.ds(base, rpt)])
            return carry

        lax.fori_loop(0, nblk, zbody, 0)
        plsc.subcore_barrier()

        def body(g, carry):
            c0 = g * K
            gathers = [
                pltpu.async_copy(y_hbm.at[sidx.at[c0 + b]], rows.at[b], sem)
                for b in range(K)
            ]
            for d in gathers:
                d.wait()
            scatters = [
                pltpu.async_copy(rows.at[b], agg.at[didx.at[c0 + b]],
                                 ssem, add=True)
                for b in range(K)
            ]
            for d in scatters:
                d.wait()
            return carry

        for h in range(halves):
            @pl.when(on0)
            def _():
                pltpu.sync_copy(src0_hbm.at[sid, pl.ds(h * ch, ch)], sidx)
                pltpu.sync_copy(dst0_hbm.at[sid, pl.ds(h * ch, ch)], didx)

            r0 = jnp.clip(c0_real - sid * cpt0 - h * ch, 0, ch)
            g0 = -(-r0 // K)
            if h == 0:
                @pl.when(jnp.logical_not(on0))
                def _():
                    pltpu.sync_copy(src1_hbm.at[sid],
                                    sidx.at[pl.ds(0, cpt1)])
                    pltpu.sync_copy(dst1_hbm.at[sid],
                                    didx.at[pl.ds(0, cpt1)])

                r1 = jnp.clip(c1_real - sid * cpt1, 0, cpt1)
                n_groups = jnp.where(on0, g0, 0 if skip1 else -(-r1 // K))
            else:
                n_groups = jnp.where(on0, g0, 0)
            lax.fori_loop(0, n_groups, body, 0)

        plsc.subcore_barrier()

        def dbody(i, carry):
            pltpu.sync_copy(agg.at[pl.ds(base, rpt)],
                            out_hbm.at[cid, pl.ds(base, rpt)])
            return carry

        lax.fori_loop(0, nblk, dbody, 0)

    return seg_kernel(y, ep["src0"], ep["dst0"], ep["src1"], ep["dst1"],
                      zeros), (1 if skip1 else NC)


def _tc_proj_first(x, wl, ones_bias, wr, br):
    R = x.shape[0]
    D = wl.shape[1]

    def body(x_ref, wl_ref, ob_ref, wr_ref, br_ref, y_ref, yr_ref):
        xv = x_ref[...]
        y_ref[...] = (jnp.dot(xv, wl_ref[...],
                              preferred_element_type=jnp.float32)
                      + ob_ref[...][None, :])
        yr_ref[...] = (jnp.dot(xv, wr_ref[...],
                               preferred_element_type=jnp.float32)
                       + br_ref[...][None, :])

    return pl.pallas_call(
        body,
        out_shape=[jax.ShapeDtypeStruct((R, D), jnp.float32),
                   jax.ShapeDtypeStruct((R, D), jnp.float32)],
    )(x, wl, ones_bias, wr, br)


def _tc_mean_proj(p, n_parts, yr, sel, wl, wr, br):
    _, R, _ = p.shape
    D2 = wl.shape[1]

    def body(p_ref, yr_ref, sel_ref, wl_ref, wr_ref, br_ref,
             y_ref, y2r_ref, inv_ref):
        pv = p_ref[...]
        agg = pv[0] if n_parts == 1 else pv[0] + pv[1]
        cnt = jnp.dot(agg, sel_ref[...],
                      preferred_element_type=jnp.float32)
        inv = 1.0 / jnp.maximum(cnt, 1.0)
        h = jnp.maximum(agg * inv + yr_ref[...], 0.0)
        y_ref[...] = jnp.dot(h, wl_ref[...],
                             preferred_element_type=jnp.float32)
        y2r_ref[...] = (jnp.dot(h, wr_ref[...],
                                preferred_element_type=jnp.float32)
                        + br_ref[...][None, :])
        inv_ref[...] = inv

    return pl.pallas_call(
        body,
        out_shape=[jax.ShapeDtypeStruct((R, D2), jnp.float32),
                   jax.ShapeDtypeStruct((R, D2), jnp.float32),
                   jax.ShapeDtypeStruct((R, 1), jnp.float32)],
    )(p, yr, sel, wl, wr, br)


def _tc_mid(p, n_parts, yr, inv, wl, wr, br):
    _, R, _ = p.shape
    D2 = wl.shape[1]

    def body(p_ref, yr_ref, inv_ref, wl_ref, wr_ref, br_ref, y_ref, yr2_ref):
        pv = p_ref[...]
        agg = pv[0] if n_parts == 1 else pv[0] + pv[1]
        h = jnp.maximum(agg * inv_ref[...] + yr_ref[...], 0.0)
        y_ref[...] = jnp.dot(h, wl_ref[...],
                             preferred_element_type=jnp.float32)
        yr2_ref[...] = (jnp.dot(h, wr_ref[...],
                                preferred_element_type=jnp.float32)
                        + br_ref[...][None, :])

    return pl.pallas_call(
        body,
        out_shape=[jax.ShapeDtypeStruct((R, D2), jnp.float32),
                   jax.ShapeDtypeStruct((R, D2), jnp.float32)],
    )(p, yr, inv, wl, wr, br)


def _tc_final(p, n_parts, yr, inv, w_head, b_head):
    _, R, _ = p.shape

    def body(p_ref, yr_ref, inv_ref, wh_ref, bh_ref, o_ref):
        pv = p_ref[...]
        agg = pv[0] if n_parts == 1 else pv[0] + pv[1]
        h = jnp.maximum(agg * inv_ref[...] + yr_ref[...], 0.0)
        o_ref[...] = (jnp.dot(h, wh_ref[...],
                              preferred_element_type=jnp.float32)
                      + bh_ref[...][None, :])

    return pl.pallas_call(
        body,
        out_shape=jax.ShapeDtypeStruct((R, 2), jnp.float32),
    )(p, yr, inv, w_head, b_head)


def kernel(x, edge_index, W1l, W1r, b1, W2l, W2r, b2, W3l, W3r, b3,
           Wreg, breg, Wcls, bcls):
    n, d_in = x.shape
    R = _node_rows(n)
    d1 = W1l.shape[1]

    x_pad = jnp.zeros((R, d_in), jnp.float32).at[:n].set(x)
    src = edge_index[0].astype(jnp.int32)
    dst = edge_index[1].astype(jnp.int32)

    W1l_p = jnp.pad(W1l, ((0, 0), (0, _PAD_COLS)))
    ones_bias = jnp.concatenate(
        [jnp.zeros((d1,), jnp.float32), jnp.ones((_PAD_COLS,), jnp.float32)])
    W1r_p = jnp.pad(W1r, ((0, 0), (0, _PAD_COLS)))
    b1_p = jnp.pad(b1, (0, _PAD_COLS))
    sel = jnp.zeros((d1 + _PAD_COLS, 1), jnp.float32).at[d1, 0].set(1.0)
    W2l_p = jnp.pad(W2l, ((0, _PAD_COLS), (0, 0)))
    W2r_p = jnp.pad(W2r, ((0, _PAD_COLS), (0, 0)))

    ep = _prep_edges(src, dst, n, R, _FRAC0)
    Y1, y1r = _tc_proj_first(x_pad, W1l_p, ones_bias, W1r_p, b1_p)
    p1, np1 = _sc_segsum(Y1, ep, n)
    Y2, y2r, inv = _tc_mean_proj(p1, np1, y1r, sel, W2l_p, W2r_p, b2)
    p2, np2 = _sc_segsum(Y2, ep, n)
    Y3, y3r = _tc_mid(p2, np2, y2r, inv, W3l, W3r, b3)
    p3, np3 = _sc_segsum(Y3, ep, n)

    w_head = jnp.concatenate([Wreg, Wcls], axis=1)
    b_head = jnp.concatenate([breg, bcls])
    out = _tc_final(p3, np3, y3r, inv, w_head, b_head)
    return out[:n, 0], out[:n, 1]


# speedup vs baseline: 1.0305x; 1.0305x over previous
import functools

import jax
import jax.numpy as jnp
from jax import lax
from jax.experimental import pallas as pl
from jax.experimental.pallas import tpu as pltpu
from jax.experimental.pallas import tpu_sc as plsc

_CHUNK = 128
_K = 4
_PAD_COLS = 16
_FRAC0 = 0.5


def _node_rows(n):
    return ((n + 1 + 31) // 32) * 32


def _prep_edges(src, dst, n, R, frac0):
    E = src.shape[0]
    mesh = plsc.VectorSubcoreMesh(core_axis_name="c", subcore_axis_name="s")
    NC, NS = mesh.num_cores, mesh.num_subcores
    C = -(-E // _CHUNK)
    cpt0 = -(-int(C * frac0) // (NS * _K)) * _K
    cpt1 = max(_K, -(-(C - NS * cpt0) // (NS * _K)) * _K)
    skip1 = NS * cpt0 >= C
    halves = 2 if cpt0 > 144 else 1
    cpt0 = -(-cpt0 // (halves * _K)) * (halves * _K)
    E_pad = NS * (cpt0 + cpt1) * _CHUNK
    n_dummy = E_pad - E
    src = jnp.concatenate([src, jnp.zeros((n_dummy,), jnp.int32)])
    dst = jnp.concatenate(
        [dst, n + (jnp.arange(n_dummy, dtype=jnp.int32) % (R - n))])
    e0 = NS * cpt0 * _CHUNK
    c0_real = min(C, NS * cpt0)
    return dict(
        src0=src[:e0].reshape(NS, cpt0, _CHUNK),
        dst0=dst[:e0].reshape(NS, cpt0, _CHUNK),
        src1=src[e0:].reshape(NS, cpt1, _CHUNK),
        dst1=dst[e0:].reshape(NS, cpt1, _CHUNK),
        cpt0=cpt0, cpt1=cpt1, c0_real=c0_real, c1_real=C - c0_real,
        skip1=skip1, halves=halves, mesh=mesh, NC=NC, NS=NS,
    )


def _sc_segsum(y, ep, n):
    R, D = y.shape
    mesh, NC, NS = ep["mesh"], ep["NC"], ep["NS"]
    K = 4 if D > 32 else 8
    cpt0, cpt1 = ep["cpt0"], ep["cpt1"]
    c0_real, c1_real = ep["c0_real"], ep["c1_real"]
    skip1, halves = ep["skip1"], ep["halves"]
    ch = cpt0 // halves
    zeros = jnp.zeros((R // NS, D), jnp.float32)
    rpt = R // NS

    @functools.partial(
        pl.kernel,
        out_type=jax.ShapeDtypeStruct((NC, R, D), jnp.float32),
        mesh=mesh,
        scratch_types=[
            pltpu.VMEM((ch, _CHUNK), jnp.int32),
            pltpu.VMEM((ch, _CHUNK), jnp.int32),
            pltpu.VMEM((K, _CHUNK, D), jnp.float32),
            pltpu.VMEM_SHARED((R, D), jnp.float32),
            pltpu.SemaphoreType.DMA,
            pltpu.SemaphoreType.DMA,
        ],
        compiler_params=pltpu.CompilerParams(use_tc_tiling_on_sc=False),
    )
    def seg_kernel(y_hbm, src0_hbm, dst0_hbm, src1_hbm, dst1_hbm, z_hbm,
                   out_hbm, sidx, didx, rows, agg, sem, ssem):
        cid = lax.axis_index("c")
        sid = lax.axis_index("s")
        base = sid * rpt
        on0 = cid == 0

        nblk = jnp.where(on0, 1, 0 if skip1 else 1)

        def zbody(i, carry):
            pltpu.sync_copy(z_hbm, agg.at[pl.ds(base, rpt)])
            return carry

        lax.fori_loop(0, nblk, zbody, 0)
        plsc.subcore_barrier()

        def body(g, carry):
            c0 = g * K
            gathers = [
                pltpu.async_copy(y_hbm.at[sidx.at[c0 + b]], rows.at[b], sem)
                for b in range(K)
            ]
            for d in gathers:
                d.wait()
            scatters = [
                pltpu.async_copy(rows.at[b], agg.at[didx.at[c0 + b]],
                                 ssem, add=True)
                for b in range(K)
            ]
            for d in scatters:
                d.wait()
            return carry

        for h in range(halves):
            @pl.when(on0)
            def _():
                pltpu.sync_copy(src0_hbm.at[sid, pl.ds(h * ch, ch)], sidx)
                pltpu.sync_copy(dst0_hbm.at[sid, pl.ds(h * ch, ch)], didx)

            r0 = jnp.clip(c0_real - sid * cpt0 - h * ch, 0, ch)
            g0 = -(-r0 // K)
            if h == 0:
                @pl.when(jnp.logical_not(on0))
                def _():
                    pltpu.sync_copy(src1_hbm.at[sid],
                                    sidx.at[pl.ds(0, cpt1)])
                    pltpu.sync_copy(dst1_hbm.at[sid],
                                    didx.at[pl.ds(0, cpt1)])

                r1 = jnp.clip(c1_real - sid * cpt1, 0, cpt1)
                n_groups = jnp.where(on0, g0, 0 if skip1 else -(-r1 // K))
            else:
                n_groups = jnp.where(on0, g0, 0)
            lax.fori_loop(0, n_groups, body, 0)

        plsc.subcore_barrier()

        def dbody(i, carry):
            pltpu.sync_copy(agg.at[pl.ds(base, rpt)],
                            out_hbm.at[cid, pl.ds(base, rpt)])
            return carry

        lax.fori_loop(0, nblk, dbody, 0)

    return seg_kernel(y, ep["src0"], ep["dst0"], ep["src1"], ep["dst1"],
                      zeros), (1 if skip1 else NC)


def _tc_proj_first(x, wl, ones_bias, wr, br):
    R = x.shape[0]
    D = wl.shape[1]

    def body(x_ref, wl_ref, ob_ref, wr_ref, br_ref, y_ref, yr_ref):
        xv = x_ref[...]
        y_ref[...] = (jnp.dot(xv, wl_ref[...],
                              preferred_element_type=jnp.float32)
                      + ob_ref[...][None, :])
        yr_ref[...] = (jnp.dot(xv, wr_ref[...],
                               preferred_element_type=jnp.float32)
                       + br_ref[...][None, :])

    return pl.pallas_call(
        body,
        out_shape=[jax.ShapeDtypeStruct((R, D), jnp.float32),
                   jax.ShapeDtypeStruct((R, D), jnp.float32)],
    )(x, wl, ones_bias, wr, br)


def _tc_mean_proj(p, n_parts, yr, sel, wl, wr, br):
    _, R, _ = p.shape
    D2 = wl.shape[1]

    def body(p_ref, yr_ref, sel_ref, wl_ref, wr_ref, br_ref,
             y_ref, y2r_ref, inv_ref):
        pv = p_ref[...]
        agg = pv[0] if n_parts == 1 else pv[0] + pv[1]
        cnt = jnp.dot(agg, sel_ref[...],
                      preferred_element_type=jnp.float32)
        inv = 1.0 / jnp.maximum(cnt, 1.0)
        h = jnp.maximum(agg * inv + yr_ref[...], 0.0)
        y_ref[...] = jnp.dot(h, wl_ref[...],
                             preferred_element_type=jnp.float32)
        y2r_ref[...] = (jnp.dot(h, wr_ref[...],
                                preferred_element_type=jnp.float32)
                        + br_ref[...][None, :])
        inv_ref[...] = inv

    return pl.pallas_call(
        body,
        out_shape=[jax.ShapeDtypeStruct((R, D2), jnp.float32),
                   jax.ShapeDtypeStruct((R, D2), jnp.float32),
                   jax.ShapeDtypeStruct((R, 1), jnp.float32)],
    )(p, yr, sel, wl, wr, br)


def _tc_mid(p, n_parts, yr, inv, wl, wr, br):
    _, R, _ = p.shape
    D2 = wl.shape[1]

    def body(p_ref, yr_ref, inv_ref, wl_ref, wr_ref, br_ref, y_ref, yr2_ref):
        pv = p_ref[...]
        agg = pv[0] if n_parts == 1 else pv[0] + pv[1]
        h = jnp.maximum(agg * inv_ref[...] + yr_ref[...], 0.0)
        y_ref[...] = jnp.dot(h, wl_ref[...],
                             preferred_element_type=jnp.float32)
        yr2_ref[...] = (jnp.dot(h, wr_ref[...],
                                preferred_element_type=jnp.float32)
                        + br_ref[...][None, :])

    return pl.pallas_call(
        body,
        out_shape=[jax.ShapeDtypeStruct((R, D2), jnp.float32),
                   jax.ShapeDtypeStruct((R, D2), jnp.float32)],
    )(p, yr, inv, wl, wr, br)


def _tc_final(p, n_parts, yr, inv, w_head, b_head):
    _, R, _ = p.shape

    def body(p_ref, yr_ref, inv_ref, wh_ref, bh_ref, o_ref):
        pv = p_ref[...]
        agg = pv[0] if n_parts == 1 else pv[0] + pv[1]
        h = jnp.maximum(agg * inv_ref[...] + yr_ref[...], 0.0)
        o_ref[...] = (jnp.dot(h, wh_ref[...],
                              preferred_element_type=jnp.float32)
                      + bh_ref[...][None, :])

    return pl.pallas_call(
        body,
        out_shape=jax.ShapeDtypeStruct((R, 2), jnp.float32),
    )(p, yr, inv, w_head, b_head)


def kernel(x, edge_index, W1l, W1r, b1, W2l, W2r, b2, W3l, W3r, b3,
           Wreg, breg, Wcls, bcls):
    n, d_in = x.shape
    R = _node_rows(n)
    d1 = W1l.shape[1]

    x_pad = jnp.zeros((R, d_in), jnp.float32).at[:n].set(x)
    src = edge_index[0].astype(jnp.int32)
    dst = edge_index[1].astype(jnp.int32)

    W1l_p = jnp.pad(W1l, ((0, 0), (0, _PAD_COLS)))
    ones_bias = jnp.concatenate(
        [jnp.zeros((d1,), jnp.float32), jnp.ones((_PAD_COLS,), jnp.float32)])
    W1r_p = jnp.pad(W1r, ((0, 0), (0, _PAD_COLS)))
    b1_p = jnp.pad(b1, (0, _PAD_COLS))
    sel = jnp.zeros((d1 + _PAD_COLS, 1), jnp.float32).at[d1, 0].set(1.0)
    W2l_p = jnp.pad(W2l, ((0, _PAD_COLS), (0, 0)))
    W2r_p = jnp.pad(W2r, ((0, _PAD_COLS), (0, 0)))

    ep = _prep_edges(src, dst, n, R, _FRAC0)
    Y1, y1r = _tc_proj_first(x_pad, W1l_p, ones_bias, W1r_p, b1_p)
    p1, np1 = _sc_segsum(Y1, ep, n)
    Y2, y2r, inv = _tc_mean_proj(p1, np1, y1r, sel, W2l_p, W2r_p, b2)
    p2, np2 = _sc_segsum(Y2, ep, n)
    Y3, y3r = _tc_mid(p2, np2, y2r, inv, W3l, W3r, b3)
    p3, np3 = _sc_segsum(Y3, ep, n)

    w_head = jnp.concatenate([Wreg, Wcls], axis=1)
    b_head = jnp.concatenate([breg, bcls])
    out = _tc_final(p3, np3, y3r, inv, w_head, b_head)
    return out[:n, 0], out[:n, 1]


# speedup vs baseline: 1.1508x; 1.1167x over previous
import functools

import jax
import jax.numpy as jnp
from jax import lax
from jax.experimental import pallas as pl
from jax.experimental.pallas import tpu as pltpu
from jax.experimental.pallas import tpu_sc as plsc

_CHUNK = 128
_K = 4
_PAD_COLS = 16
_FRAC0 = 0.5


def _node_rows(n):
    return ((n + 1 + 31) // 32) * 32


def _prep_edges(src, dst, n, R, frac0):
    E = src.shape[0]
    mesh = plsc.VectorSubcoreMesh(core_axis_name="c", subcore_axis_name="s")
    NC, NS = mesh.num_cores, mesh.num_subcores
    C = -(-E // _CHUNK)
    cpt0 = -(-int(C * frac0) // (NS * _K)) * _K
    cpt1 = max(_K, -(-(C - NS * cpt0) // (NS * _K)) * _K)
    skip1 = NS * cpt0 >= C
    halves = 2 if cpt0 > 144 else 1
    cpt0 = -(-cpt0 // (halves * _K)) * (halves * _K)
    E_pad = NS * (cpt0 + cpt1) * _CHUNK
    n_dummy = E_pad - E
    src = jnp.concatenate([src, jnp.zeros((n_dummy,), jnp.int32)])
    dst = jnp.concatenate(
        [dst, n + (jnp.arange(n_dummy, dtype=jnp.int32) % (R - n))])
    e0 = NS * cpt0 * _CHUNK
    c0_real = min(C, NS * cpt0)
    return dict(
        src0=src[:e0].reshape(NS, cpt0, _CHUNK),
        dst0=dst[:e0].reshape(NS, cpt0, _CHUNK),
        src1=src[e0:].reshape(NS, cpt1, _CHUNK),
        dst1=dst[e0:].reshape(NS, cpt1, _CHUNK),
        cpt0=cpt0, cpt1=cpt1, c0_real=c0_real, c1_real=C - c0_real,
        skip1=skip1, halves=halves, mesh=mesh, NC=NC, NS=NS,
    )


def _sc_segsum(y, ep, n):
    R, D = y.shape
    mesh, NC, NS = ep["mesh"], ep["NC"], ep["NS"]
    K = 4 if D > 32 else 8
    cpt0, cpt1 = ep["cpt0"], ep["cpt1"]
    c0_real, c1_real = ep["c0_real"], ep["c1_real"]
    skip1, halves = ep["skip1"], ep["halves"]
    ch = cpt0 // halves
    zeros = jnp.zeros((R // NS, D), jnp.float32)
    rpt = R // NS

    @functools.partial(
        pl.kernel,
        out_type=jax.ShapeDtypeStruct((NC, R, D), jnp.float32),
        mesh=mesh,
        scratch_types=[
            pltpu.VMEM((ch, _CHUNK), jnp.int32),
            pltpu.VMEM((ch, _CHUNK), jnp.int32),
            pltpu.VMEM((K, _CHUNK, D), jnp.float32),
            pltpu.VMEM_SHARED((R, D), jnp.float32),
            pltpu.SemaphoreType.DMA,
            pltpu.SemaphoreType.DMA,
        ],
        compiler_params=pltpu.CompilerParams(use_tc_tiling_on_sc=False),
    )
    def seg_kernel(y_hbm, src0_hbm, dst0_hbm, src1_hbm, dst1_hbm, z_hbm,
                   out_hbm, sidx, didx, rows, agg, sem, ssem):
        cid = lax.axis_index("c")
        sid = lax.axis_index("s")
        base = sid * rpt
        on0 = cid == 0

        nblk = jnp.where(on0, 1, 0 if skip1 else 1)

        def zbody(i, carry):
            pltpu.sync_copy(z_hbm, agg.at[pl.ds(base, rpt)])
            return carry

        lax.fori_loop(0, nblk, zbody, 0)
        plsc.subcore_barrier()

        def body(g, carry):
            c0 = g * K
            gathers = [
                pltpu.async_copy(y_hbm.at[sidx.at[c0 + b]], rows.at[b], sem)
                for b in range(K)
            ]
            scatters = []
            for b in range(K):
                gathers[b].wait()
                scatters.append(
                    pltpu.async_copy(rows.at[b], agg.at[didx.at[c0 + b]],
                                     ssem, add=True))
            for d in scatters:
                d.wait()
            return carry

        for h in range(halves):
            @pl.when(on0)
            def _():
                pltpu.sync_copy(src0_hbm.at[sid, pl.ds(h * ch, ch)], sidx)
                pltpu.sync_copy(dst0_hbm.at[sid, pl.ds(h * ch, ch)], didx)

            r0 = jnp.clip(c0_real - sid * cpt0 - h * ch, 0, ch)
            g0 = -(-r0 // K)
            if h == 0:
                @pl.when(jnp.logical_not(on0))
                def _():
                    pltpu.sync_copy(src1_hbm.at[sid],
                                    sidx.at[pl.ds(0, cpt1)])
                    pltpu.sync_copy(dst1_hbm.at[sid],
                                    didx.at[pl.ds(0, cpt1)])

                r1 = jnp.clip(c1_real - sid * cpt1, 0, cpt1)
                n_groups = jnp.where(on0, g0, 0 if skip1 else -(-r1 // K))
            else:
                n_groups = jnp.where(on0, g0, 0)
            lax.fori_loop(0, n_groups, body, 0)

        plsc.subcore_barrier()

        def dbody(i, carry):
            pltpu.sync_copy(agg.at[pl.ds(base, rpt)],
                            out_hbm.at[cid, pl.ds(base, rpt)])
            return carry

        lax.fori_loop(0, nblk, dbody, 0)

    return seg_kernel(y, ep["src0"], ep["dst0"], ep["src1"], ep["dst1"],
                      zeros), (1 if skip1 else NC)


def _tc_proj_first(x, wl, ones_bias, wr, br):
    R = x.shape[0]
    D = wl.shape[1]

    def body(x_ref, wl_ref, ob_ref, wr_ref, br_ref, y_ref, yr_ref):
        xv = x_ref[...]
        y_ref[...] = (jnp.dot(xv, wl_ref[...],
                              preferred_element_type=jnp.float32)
                      + ob_ref[...][None, :])
        yr_ref[...] = (jnp.dot(xv, wr_ref[...],
                               preferred_element_type=jnp.float32)
                       + br_ref[...][None, :])

    return pl.pallas_call(
        body,
        out_shape=[jax.ShapeDtypeStruct((R, D), jnp.float32),
                   jax.ShapeDtypeStruct((R, D), jnp.float32)],
    )(x, wl, ones_bias, wr, br)


def _tc_mean_proj(p, n_parts, yr, sel, wl, wr, br):
    _, R, _ = p.shape
    D2 = wl.shape[1]

    def body(p_ref, yr_ref, sel_ref, wl_ref, wr_ref, br_ref,
             y_ref, y2r_ref, inv_ref):
        pv = p_ref[...]
        agg = pv[0] if n_parts == 1 else pv[0] + pv[1]
        cnt = jnp.dot(agg, sel_ref[...],
                      preferred_element_type=jnp.float32)
        inv = 1.0 / jnp.maximum(cnt, 1.0)
        h = jnp.maximum(agg * inv + yr_ref[...], 0.0)
        y_ref[...] = jnp.dot(h, wl_ref[...],
                             preferred_element_type=jnp.float32)
        y2r_ref[...] = (jnp.dot(h, wr_ref[...],
                                preferred_element_type=jnp.float32)
                        + br_ref[...][None, :])
        inv_ref[...] = inv

    return pl.pallas_call(
        body,
        out_shape=[jax.ShapeDtypeStruct((R, D2), jnp.float32),
                   jax.ShapeDtypeStruct((R, D2), jnp.float32),
                   jax.ShapeDtypeStruct((R, 1), jnp.float32)],
    )(p, yr, sel, wl, wr, br)


def _tc_mid(p, n_parts, yr, inv, wl, wr, br):
    _, R, _ = p.shape
    D2 = wl.shape[1]

    def body(p_ref, yr_ref, inv_ref, wl_ref, wr_ref, br_ref, y_ref, yr2_ref):
        pv = p_ref[...]
        agg = pv[0] if n_parts == 1 else pv[0] + pv[1]
        h = jnp.maximum(agg * inv_ref[...] + yr_ref[...], 0.0)
        y_ref[...] = jnp.dot(h, wl_ref[...],
                             preferred_element_type=jnp.float32)
        yr2_ref[...] = (jnp.dot(h, wr_ref[...],
                                preferred_element_type=jnp.float32)
                        + br_ref[...][None, :])

    return pl.pallas_call(
        body,
        out_shape=[jax.ShapeDtypeStruct((R, D2), jnp.float32),
                   jax.ShapeDtypeStruct((R, D2), jnp.float32)],
    )(p, yr, inv, wl, wr, br)


def _tc_final(p, n_parts, yr, inv, w_head, b_head):
    _, R, _ = p.shape

    def body(p_ref, yr_ref, inv_ref, wh_ref, bh_ref, o_ref):
        pv = p_ref[...]
        agg = pv[0] if n_parts == 1 else pv[0] + pv[1]
        h = jnp.maximum(agg * inv_ref[...] + yr_ref[...], 0.0)
        o_ref[...] = (jnp.dot(h, wh_ref[...],
                              preferred_element_type=jnp.float32)
                      + bh_ref[...][None, :])

    return pl.pallas_call(
        body,
        out_shape=jax.ShapeDtypeStruct((R, 2), jnp.float32),
    )(p, yr, inv, w_head, b_head)


def kernel(x, edge_index, W1l, W1r, b1, W2l, W2r, b2, W3l, W3r, b3,
           Wreg, breg, Wcls, bcls):
    n, d_in = x.shape
    R = _node_rows(n)
    d1 = W1l.shape[1]

    x_pad = jnp.zeros((R, d_in), jnp.float32).at[:n].set(x)
    src = edge_index[0].astype(jnp.int32)
    dst = edge_index[1].astype(jnp.int32)

    W1l_p = jnp.pad(W1l, ((0, 0), (0, _PAD_COLS)))
    ones_bias = jnp.concatenate(
        [jnp.zeros((d1,), jnp.float32), jnp.ones((_PAD_COLS,), jnp.float32)])
    W1r_p = jnp.pad(W1r, ((0, 0), (0, _PAD_COLS)))
    b1_p = jnp.pad(b1, (0, _PAD_COLS))
    sel = jnp.zeros((d1 + _PAD_COLS, 1), jnp.float32).at[d1, 0].set(1.0)
    W2l_p = jnp.pad(W2l, ((0, _PAD_COLS), (0, 0)))
    W2r_p = jnp.pad(W2r, ((0, _PAD_COLS), (0, 0)))

    ep = _prep_edges(src, dst, n, R, _FRAC0)
    Y1, y1r = _tc_proj_first(x_pad, W1l_p, ones_bias, W1r_p, b1_p)
    p1, np1 = _sc_segsum(Y1, ep, n)
    Y2, y2r, inv = _tc_mean_proj(p1, np1, y1r, sel, W2l_p, W2r_p, b2)
    p2, np2 = _sc_segsum(Y2, ep, n)
    Y3, y3r = _tc_mid(p2, np2, y2r, inv, W3l, W3r, b3)
    p3, np3 = _sc_segsum(Y3, ep, n)

    w_head = jnp.concatenate([Wreg, Wcls], axis=1)
    b_head = jnp.concatenate([breg, bcls])
    out = _tc_final(p3, np3, y3r, inv, w_head, b_head)
    return out[:n, 0], out[:n, 1]
